# baseline (speedup 1.0000x reference)
import functools

import jax
import jax.numpy as jnp
from jax import lax
from jax.experimental import pallas as pl
from jax.experimental.pallas import tpu as pltpu
from jax.experimental.pallas import tpu_sc as plsc

DIM = 32
NROWS = 1000000
BATCH = 16384
NC = 2
NS = 16
L = 16
NW = NC * NS
BPW = BATCH // NW
Q = 128
NQ = BPW // Q
PACK = 128 // DIM
NPK = NROWS // PACK

SB = 512
NSB_FULL = NROWS // SB
SB_TAIL = NROWS - NSB_FULL * SB
TAIL_WID = NSB_FULL % NW


def _repack_body(uembT_hbm, iembT_hbm, upk_hbm, ipk_hbm,
                 nat0, nat1, pk0, pk1, sem_in0, sem_in1, sem_out0, sem_out1):
    wid = lax.axis_index("s") * NC + lax.axis_index("c")
    nats = [nat0, nat1]
    pks = [pk0, pk1]
    sin = [sem_in0, sem_in1]
    sout = [sem_out0, sem_out1]
    iota = lax.iota(jnp.int32, L)

    def run_table(src_hbm, dst_hbm):
        def fire_in(s, p):
            cs = []
            for i in range(DIM // 8):
                cs.append(pltpu.async_copy(
                    src_hbm.at[pl.ds(8 * i, 8), pl.ds(s * SB, SB)],
                    nats[p].at[pl.ds(8 * i, 8)], sin[p]))
            return cs

        def shuffle(p, nb):
            def col(c, carry):
                b16 = c * L + iota
                row = b16 >> 2
                colb = (b16 & (PACK - 1)) * DIM
                for d in range(DIM):
                    plsc.store_scatter(pks[p], [row, colb + d],
                                       nats[p][d, pl.ds(c * L, L)])
                return carry

            lax.fori_loop(0, nb // L, col, 0)

        def fire_out(s, p, nrow):
            return pltpu.async_copy(
                pks[p].at[pl.ds(0, nrow)],
                dst_hbm.at[pl.ds(s * (SB // PACK), nrow)], sout[p])

        nsb_max = (NSB_FULL + NW - 1) // NW

        def step(k, carry):
            s = k * NW + wid

            @pl.when(s < NSB_FULL)
            def _():
                cs = fire_in(s, 0)
                for c in cs:
                    c.wait()
                shuffle(0, SB)
                fire_out(s, 0, SB // PACK).wait()

            return carry

        lax.fori_loop(0, nsb_max, step, 0)

    run_table(uembT_hbm, upk_hbm)
    run_table(iembT_hbm, ipk_hbm)

    @pl.when(wid == TAIL_WID)
    def _():
        s = NSB_FULL
        for src_hbm, dst_hbm, p in ((uembT_hbm, upk_hbm, 0),
                                    (iembT_hbm, ipk_hbm, 1)):
            cs = []
            for i in range(DIM // 8):
                cs.append(pltpu.async_copy(
                    src_hbm.at[pl.ds(8 * i, 8), pl.ds(s * SB, SB_TAIL)],
                    nats[p].at[pl.ds(8 * i, 8), pl.ds(0, SB_TAIL)], sin[p]))
            for c in cs:
                c.wait()

            def col(c2, carry):
                b16 = c2 * L + iota
                row = b16 >> 2
                colb = (b16 & (PACK - 1)) * DIM
                for d in range(DIM):
                    plsc.store_scatter(pks[p], [row, colb + d],
                                       nats[p][d, pl.ds(c2 * L, L)])
                return carry

            lax.fori_loop(0, SB_TAIL // L, col, 0)
            pltpu.async_copy(
                pks[p].at[pl.ds(0, SB_TAIL // PACK)],
                dst_hbm.at[pl.ds(s * (SB // PACK), SB_TAIL // PACK)],
                sout[p]).wait()


def _score_body(uid_hbm, iid_hbm, upack_hbm, ipack_hbm, ub_hbm, ib_hbm,
                out_hbm, uid_v, iid_v, uk_v, ik_v, urow0, urow1, irow0, irow1,
                ubv, ibv, dotv, sem_u0, sem_u1, sem_i0, sem_i1, sem_ub,
                sem_ib):
    wid = lax.axis_index("s") * NC + lax.axis_index("c")
    base = wid * BPW

    pltpu.sync_copy(uid_hbm.at[pl.ds(base, BPW)], uid_v)
    pltpu.sync_copy(iid_hbm.at[pl.ds(base, BPW)], iid_v)

    cub = pltpu.async_copy(ub_hbm.at[uid_v], ubv, sem_ub)
    cib = pltpu.async_copy(ib_hbm.at[iid_v], ibv, sem_ib)

    def mkidx(c, carry):
        o = c * L
        uk_v[pl.ds(o, L)] = uid_v[pl.ds(o, L)] >> 2
        ik_v[pl.ds(o, L)] = iid_v[pl.ds(o, L)] >> 2
        return carry

    lax.fori_loop(0, BPW // L, mkidx, 0)

    ubufs = [urow0, urow1]
    ibufs = [irow0, irow1]
    usems = [sem_u0, sem_u1]
    isems = [sem_i0, sem_i1]

    def fire(q):
        b = q % 2
        cu = pltpu.async_copy(
            upack_hbm.at[uk_v.at[pl.ds(q * Q, Q)]], ubufs[b], usems[b])
        ci = pltpu.async_copy(
            ipack_hbm.at[ik_v.at[pl.ds(q * Q, Q)]], ibufs[b], isems[b])
        return cu, ci

    iota = lax.iota(jnp.int32, L)

    def extract(q):
        b = q % 2
        ub, ib = ubufs[b], ibufs[b]
        for g in range(Q // L):
            o = q * Q + g * L
            u16 = uid_v[pl.ds(o, L)]
            i16 = iid_v[pl.ds(o, L)]
            row = g * L + iota
            ucol0 = (u16 & (PACK - 1)) * DIM
            icol0 = (i16 & (PACK - 1)) * DIM
            acc = ubv[pl.ds(o, L)] + ibv[pl.ds(o, L)]
            for d in range(DIM):
                acc = acc + (plsc.load_gather(ub, [row, ucol0 + d])
                             * plsc.load_gather(ib, [row, icol0 + d]))
            dotv[pl.ds(o, L)] = acc

    pend = [fire(0), fire(1)]
    cub.wait()
    cib.wait()
    for q in range(NQ):
        cu, ci = pend[q % 2]
        cu.wait()
        ci.wait()
        extract(q)
        if q + 2 < NQ:
            pend[q % 2] = fire(q + 2)

    pltpu.sync_copy(dotv, out_hbm.at[pl.ds(base, BPW)])


@jax.jit
def kernel(user_ids, item_ids, user_emb, item_emb, user_bias, item_bias):
    uid = user_ids.astype(jnp.int32)
    iid = item_ids.astype(jnp.int32)
    mesh = plsc.VectorSubcoreMesh(core_axis_name="c", subcore_axis_name="s")
    params = pltpu.CompilerParams(
        needs_layout_passes=False, use_tc_tiling_on_sc=False)

    repack = functools.partial(
        pl.kernel,
        mesh=mesh,
        compiler_params=params,
        out_type=(jax.ShapeDtypeStruct((NPK, 128), jnp.float32),
                  jax.ShapeDtypeStruct((NPK, 128), jnp.float32)),
        scratch_types=[
            pltpu.VMEM((DIM, SB), jnp.float32),
            pltpu.VMEM((DIM, SB), jnp.float32),
            pltpu.VMEM((SB // PACK, 128), jnp.float32),
            pltpu.VMEM((SB // PACK, 128), jnp.float32),
            pltpu.SemaphoreType.DMA,
            pltpu.SemaphoreType.DMA,
            pltpu.SemaphoreType.DMA,
            pltpu.SemaphoreType.DMA,
        ],
    )(_repack_body)
    upk, ipk = repack(user_emb.T, item_emb.T)

    score = functools.partial(
        pl.kernel,
        mesh=mesh,
        compiler_params=params,
        out_type=jax.ShapeDtypeStruct((BATCH,), jnp.float32),
        scratch_types=[
            pltpu.VMEM((BPW,), jnp.int32),
            pltpu.VMEM((BPW,), jnp.int32),
            pltpu.VMEM((BPW,), jnp.int32),
            pltpu.VMEM((BPW,), jnp.int32),
            pltpu.VMEM((Q, 128), jnp.float32),
            pltpu.VMEM((Q, 128), jnp.float32),
            pltpu.VMEM((Q, 128), jnp.float32),
            pltpu.VMEM((Q, 128), jnp.float32),
            pltpu.VMEM((BPW,), jnp.float32),
            pltpu.VMEM((BPW,), jnp.float32),
            pltpu.VMEM((BPW,), jnp.float32),
            pltpu.SemaphoreType.DMA,
            pltpu.SemaphoreType.DMA,
            pltpu.SemaphoreType.DMA,
            pltpu.SemaphoreType.DMA,
            pltpu.SemaphoreType.DMA,
            pltpu.SemaphoreType.DMA,
        ],
    )(_score_body)
    return score(uid, iid, upk, ipk,
                 user_bias.reshape(-1), item_bias.reshape(-1))


# speedup vs baseline: 7.2359x; 7.2359x over previous
import functools

import jax
import jax.numpy as jnp
from jax import lax
from jax.experimental import pallas as pl
from jax.experimental.pallas import tpu as pltpu
from jax.experimental.pallas import tpu_sc as plsc

DIM = 32
BATCH = 16384
NC = 2
NS = 16
L = 16
NW = NC * NS
BPW = BATCH // NW
Q = 128
NQ = BPW // Q
PACK = 128 // DIM


def _bpr_body(uid_hbm, iid_hbm, upack_hbm, ipack_hbm, ub_hbm, ib_hbm, out_hbm,
              uid_v, iid_v, uk_v, ik_v, urow0, urow1, irow0, irow1,
              ubv, ibv, dotv, sem_u0, sem_u1, sem_i0, sem_i1, sem_ub, sem_ib):
    wid = lax.axis_index("s") * NC + lax.axis_index("c")
    base = wid * BPW

    pltpu.sync_copy(uid_hbm.at[pl.ds(base, BPW)], uid_v)
    pltpu.sync_copy(iid_hbm.at[pl.ds(base, BPW)], iid_v)

    cub = pltpu.async_copy(ub_hbm.at[uid_v], ubv, sem_ub)
    cib = pltpu.async_copy(ib_hbm.at[iid_v], ibv, sem_ib)

    def mkidx(c, carry):
        o = c * L
        uk_v[pl.ds(o, L)] = uid_v[pl.ds(o, L)] >> 2
        ik_v[pl.ds(o, L)] = iid_v[pl.ds(o, L)] >> 2
        return carry

    lax.fori_loop(0, BPW // L, mkidx, 0)

    ubufs = [urow0, urow1]
    ibufs = [irow0, irow1]
    usems = [sem_u0, sem_u1]
    isems = [sem_i0, sem_i1]

    def fire(q):
        b = q % 2
        cu = pltpu.async_copy(
            upack_hbm.at[uk_v.at[pl.ds(q * Q, Q)]], ubufs[b], usems[b])
        ci = pltpu.async_copy(
            ipack_hbm.at[ik_v.at[pl.ds(q * Q, Q)]], ibufs[b], isems[b])
        return cu, ci

    iota = lax.iota(jnp.int32, L)

    def extract(q):
        b = q % 2
        ub, ib = ubufs[b], ibufs[b]
        for g in range(Q // L):
            o = q * Q + g * L
            u16 = uid_v[pl.ds(o, L)]
            i16 = iid_v[pl.ds(o, L)]
            row = g * L + iota
            ucol0 = (u16 & (PACK - 1)) * DIM
            icol0 = (i16 & (PACK - 1)) * DIM
            acc = ubv[pl.ds(o, L)] + ibv[pl.ds(o, L)]
            for d in range(DIM):
                acc = acc + (plsc.load_gather(ub, [row, ucol0 + d])
                             * plsc.load_gather(ib, [row, icol0 + d]))
            dotv[pl.ds(o, L)] = acc

    pend = [fire(0), fire(1)]
    cub.wait()
    cib.wait()
    for q in range(NQ):
        cu, ci = pend[q % 2]
        cu.wait()
        ci.wait()
        extract(q)
        if q + 2 < NQ:
            pend[q % 2] = fire(q + 2)

    pltpu.sync_copy(dotv, out_hbm.at[pl.ds(base, BPW)])


@jax.jit
def kernel(user_ids, item_ids, user_emb, item_emb, user_bias, item_bias):
    uid = user_ids.astype(jnp.int32)
    iid = item_ids.astype(jnp.int32)
    mesh = plsc.VectorSubcoreMesh(core_axis_name="c", subcore_axis_name="s")
    run = functools.partial(
        pl.kernel,
        mesh=mesh,
        compiler_params=pltpu.CompilerParams(
            needs_layout_passes=False, use_tc_tiling_on_sc=False),
        out_type=jax.ShapeDtypeStruct((BATCH,), jnp.float32),
        scratch_types=[
            pltpu.VMEM((BPW,), jnp.int32),
            pltpu.VMEM((BPW,), jnp.int32),
            pltpu.VMEM((BPW,), jnp.int32),
            pltpu.VMEM((BPW,), jnp.int32),
            pltpu.VMEM((Q, 128), jnp.float32),
            pltpu.VMEM((Q, 128), jnp.float32),
            pltpu.VMEM((Q, 128), jnp.float32),
            pltpu.VMEM((Q, 128), jnp.float32),
            pltpu.VMEM((BPW,), jnp.float32),
            pltpu.VMEM((BPW,), jnp.float32),
            pltpu.VMEM((BPW,), jnp.float32),
            pltpu.SemaphoreType.DMA,
            pltpu.SemaphoreType.DMA,
            pltpu.SemaphoreType.DMA,
            pltpu.SemaphoreType.DMA,
            pltpu.SemaphoreType.DMA,
            pltpu.SemaphoreType.DMA,
        ],
    )(_bpr_body)
    return run(uid, iid,
               user_emb.reshape(250000, 128), item_emb.reshape(250000, 128),
               user_bias.reshape(-1), item_bias.reshape(-1))
